# baseline (speedup 1.0000x reference)
import numpy as np
import jax
from jax import lax
import jax.numpy as jnp
from jax.experimental import pallas as pl
from jax.experimental.pallas import tpu as pltpu
from jax.experimental.pallas import tpu_sc as plsc

_N = 10000
_E = 320000
_P2 = 131072
_BN_S = np.float32(1.0 / np.sqrt(1.0 + 1e-5))

_NC, _NS = 2, 16
_NPAD = 10240
_EPAD = 327680
_CHUNK = 256
_RPT = _NPAD // _NS

_MESH = plsc.VectorSubcoreMesh(core_axis_name="c", subcore_axis_name="s",
                               num_cores=_NC, num_subcores=_NS)

_ECT = _EPAD // _NS
_NCHUNKS = _ECT // _CHUNK
_NPAIRS = _NCHUNKS // 2


def _half(t):
    return t.reshape(-1, 2, 64).reshape(-1, 64)


def _sc_seg_sum(tab, srcp, dstp, zrows, zcnt, ones_r, with_cnt):
    def body(tar, srcr, dstr, zr, zc, onesr, out, outc,
             idxs, idxd, rows_a, rows_b, ones_v, acc, accc,
             semi, sem_a, sem_b):
        c = lax.axis_index("c")
        s = lax.axis_index("s")
        r0 = s * _RPT
        ci = pltpu.async_copy(srcr.at[c, s], idxs, semi)
        cd = pltpu.async_copy(dstr.at[s], idxd, semi)
        pltpu.sync_copy(zr, acc.at[pl.ds(r0, _RPT)])
        if with_cnt:
            @pl.when(c == 0)
            def _():
                pltpu.sync_copy(zc, accc.at[pl.ds(r0, _RPT)])
                pltpu.sync_copy(onesr, ones_v)
        ci.wait()
        cd.wait()

        def gather(k, rows, sem):
            pltpu.async_copy(tar.at[idxs.at[k]], rows, sem)

        def consume(k, rows, sem):
            pltpu.make_async_copy(tar.at[idxs.at[k]], rows, sem).wait()
            pltpu.sync_copy(rows, acc.at[idxd.at[k]], add=True)
            if with_cnt:
                @pl.when(c == 0)
                def _():
                    pltpu.sync_copy(ones_v, accc.at[idxd.at[k]], add=True)

        gather(0, rows_a, sem_a)
        plsc.subcore_barrier()

        @pl.loop(0, _NPAIRS)
        def _(p):
            ka = 2 * p
            gather(ka + 1, rows_b, sem_b)
            consume(ka, rows_a, sem_a)

            @pl.when(p + 1 < _NPAIRS)
            def _():
                gather(ka + 2, rows_a, sem_a)

            consume(ka + 1, rows_b, sem_b)

        plsc.subcore_barrier()

        @pl.loop(0, _RPT // 128)
        def _(i):
            rr = r0 + i * 128
            pltpu.sync_copy(acc.at[pl.ds(rr, 128)], rows_a.at[pl.ds(0, 128)])
            pltpu.sync_copy(rows_a.at[pl.ds(0, 128)], out.at[c, pl.ds(rr, 128)])

        if with_cnt:
            @pl.when(c == 0)
            def _():
                @pl.loop(0, _RPT // 128)
                def _(i):
                    rr = r0 + i * 128
                    pltpu.sync_copy(accc.at[pl.ds(rr, 128)],
                                    ones_v.at[pl.ds(0, 128)])
                    pltpu.sync_copy(ones_v.at[pl.ds(0, 128)],
                                    outc.at[pl.ds(rr, 128)])

    f = pl.kernel(
        body,
        out_type=(jax.ShapeDtypeStruct((_NC, _NPAD, 64), jnp.float32),
                  jax.ShapeDtypeStruct((_NPAD, 16), jnp.float32)),
        mesh=_MESH,
        compiler_params=pltpu.CompilerParams(use_tc_tiling_on_sc=False),
        scratch_types=[
            pltpu.VMEM((_NCHUNKS, _CHUNK), jnp.int32),
            pltpu.VMEM((_NCHUNKS, _CHUNK), jnp.int32),
            pltpu.VMEM((_CHUNK, 64), jnp.float32),
            pltpu.VMEM((_CHUNK, 64), jnp.float32),
            pltpu.VMEM((_CHUNK, 16), jnp.float32),
            pltpu.VMEM_SHARED((_NPAD, 64), jnp.float32),
            pltpu.VMEM_SHARED((_NPAD, 16), jnp.float32),
            pltpu.SemaphoreType.DMA,
            pltpu.SemaphoreType.DMA,
            pltpu.SemaphoreType.DMA,
        ],
    )
    return f(tab, srcp, dstp, zrows, zcnt, ones_r)


_EPT_L = _P2 // (_NC * _NS)
_NCH_L = _EPT_L // _CHUNK


def _sc_link_gather(zu, zv, uidx, vidx):
    def body(tu, tv, ur, vr, gu, gv, idx_u, idx_v, rows_u, rows_v,
             semi, sem_u, sem_v):
        c = lax.axis_index("c")
        s = lax.axis_index("s")
        t = c * _NS + s
        base = t * _EPT_L
        cu = pltpu.async_copy(ur.at[t], idx_u, semi)
        cv = pltpu.async_copy(vr.at[t], idx_v, semi)
        cu.wait()
        cv.wait()
        pltpu.async_copy(tu.at[idx_u.at[0]], rows_u, sem_u)

        @pl.loop(0, _NCH_L)
        def _(k):
            off = base + k * _CHUNK
            pltpu.async_copy(tv.at[idx_v.at[k]], rows_v, sem_v)
            pltpu.make_async_copy(tu.at[idx_u.at[k]], rows_u, sem_u).wait()
            pltpu.sync_copy(rows_u, gu.at[pl.ds(off, _CHUNK)])

            @pl.when(k + 1 < _NCH_L)
            def _():
                pltpu.async_copy(tu.at[idx_u.at[k + 1]], rows_u, sem_u)

            pltpu.make_async_copy(tv.at[idx_v.at[k]], rows_v, sem_v).wait()
            pltpu.sync_copy(rows_v, gv.at[pl.ds(off, _CHUNK)])

    f = pl.kernel(
        body,
        out_type=(jax.ShapeDtypeStruct((_P2, 128), jnp.float32),
                  jax.ShapeDtypeStruct((_P2, 128), jnp.float32)),
        mesh=_MESH,
        compiler_params=pltpu.CompilerParams(use_tc_tiling_on_sc=False),
        scratch_types=[
            pltpu.VMEM((_NCH_L, _CHUNK), jnp.int32),
            pltpu.VMEM((_NCH_L, _CHUNK), jnp.int32),
            pltpu.VMEM((_CHUNK, 128), jnp.float32),
            pltpu.VMEM((_CHUNK, 128), jnp.float32),
            pltpu.SemaphoreType.DMA,
            pltpu.SemaphoreType.DMA,
            pltpu.SemaphoreType.DMA,
        ],
    )
    return f(zu, zv, uidx, vidx)


def _cnt_col(cr):
    return jnp.maximum(cr[:, :1], 1.0)


_RB = 2000
_NRB = _N // _RB


def _row_call(body, n_rowed, weight_shapes, out_cols):
    def block(c):
        return pl.BlockSpec((_RB, c), lambda i: (i, 0))

    def full(shape):
        return pl.BlockSpec(shape, lambda i: (0, 0))

    def make(*arrays):
        in_specs = [block(a.shape[1]) for a in arrays[:n_rowed]]
        in_specs += [full(s) for s in weight_shapes]
        return pl.pallas_call(
            body,
            grid=(_NRB,),
            in_specs=in_specs,
            out_specs=tuple(block(c) for c in out_cols),
            out_shape=tuple(jax.ShapeDtypeStruct((_N, c), jnp.float32)
                            for c in out_cols),
        )(*arrays)
    return make


def _phase_a(x, a0, a1, c0, Wl1T, bl1, Wr1T, g1, b1):
    def body(x_ref, a0r, a1r, c0r, wl_ref, blr, wr_ref, gr, br, hf, hb):
        mean = jnp.concatenate([a0r[...], a1r[...]], axis=-1) / _cnt_col(c0r[...])
        h = jnp.dot(mean, wl_ref[...], preferred_element_type=jnp.float32)
        h = h + jnp.dot(x_ref[...], wr_ref[...], preferred_element_type=jnp.float32)
        h = (h + blr[...]) * (gr[...] * _BN_S) + br[...]
        h = jnp.maximum(h, 0.0)
        hf[...] = h[:, :128]
        hb[...] = h[:, 128:]

    wts = [Wl1T, bl1, Wr1T, g1, b1]
    return _row_call(body, 4, [w.shape for w in wts], (128, 128))(
        x, a0, a1, c0, *wts)


def _phase_b(hf, hb, f0, f1, b0, b1, c0, Wl2T, bl2, Wr2T, g2, b2,
             Wl3T, Wr3T):
    def body(hfr, hbr, f0r, f1r, b0r, b1r, c0r, wl, blr, wr, gr, br,
             wl3, wr3, yl_ref, yr_ref):
        cnt = _cnt_col(c0r[...])
        mean = jnp.concatenate(
            [f0r[...], f1r[...], b0r[...], b1r[...]], axis=-1) / cnt
        h1 = jnp.concatenate([hfr[...], hbr[...]], axis=-1)
        h = jnp.dot(mean, wl[...], preferred_element_type=jnp.float32)
        h = h + jnp.dot(h1, wr[...], preferred_element_type=jnp.float32)
        h = (h + blr[...]) * (gr[...] * _BN_S) + br[...]
        h = jnp.maximum(h, 0.0)
        yl_ref[...] = jnp.dot(h, wl3[...], preferred_element_type=jnp.float32)
        yr_ref[...] = jnp.dot(h, wr3[...], preferred_element_type=jnp.float32)

    wts = [Wl2T, bl2, Wr2T, g2, b2, Wl3T, Wr3T]
    return _row_call(body, 7, [w.shape for w in wts], (128, 128))(
        hf, hb, f0, f1, b0, b1, c0, *wts)


def _phase_c(a0, a1, c0, yr, bl3, Wc1T, bc1, Wc2T, bc2, Wp1aT, Wp1bT, bp1):
    def body(a0r, a1r, c0r, yr_ref, bl3r, wc1, bc1r, wc2, bc2r, wpa, wpb,
             bp1r, z_ref, nl_ref, zu_ref, zv_ref):
        mean = jnp.concatenate([a0r[...], a1r[...]], axis=-1) / _cnt_col(c0r[...])
        z = mean + bl3r[...] + yr_ref[...]
        z_ref[...] = z
        t = jnp.maximum(jnp.dot(z, wc1[...], preferred_element_type=jnp.float32)
                        + bc1r[...], 0.0)
        nl_ref[...] = jnp.dot(t, wc2[...], preferred_element_type=jnp.float32) + bc2r[...]
        zu_ref[...] = jnp.dot(z, wpa[...], preferred_element_type=jnp.float32) + bp1r[...]
        zv_ref[...] = jnp.dot(z, wpb[...], preferred_element_type=jnp.float32)

    wts = [bl3, Wc1T, bc1, Wc2T, bc2, Wp1aT, Wp1bT, bp1]
    return _row_call(body, 4, [w.shape for w in wts], (128, 4, 128, 128))(
        a0, a1, c0, yr, *wts)


def _phase_d(gu, gv, wp2, bp2):
    _NB = 16
    _B = _P2 // _NB

    def body(gu_ref, gv_ref, w_ref, b_ref, out_ref):
        e = jnp.maximum(gu_ref[...] + gv_ref[...], 0.0)
        logit = jnp.sum(e * w_ref[...], axis=-1) + b_ref[0, 0]
        out_ref[...] = jax.nn.sigmoid(logit).reshape(_B // 128, 128)

    out = pl.pallas_call(
        body,
        grid=(_NB,),
        in_specs=[pl.BlockSpec((_B, 128), lambda i: (i, 0)),
                  pl.BlockSpec((_B, 128), lambda i: (i, 0)),
                  pl.BlockSpec((1, 128), lambda i: (0, 0)),
                  pl.BlockSpec((1, 1), lambda i: (0, 0))],
        out_specs=pl.BlockSpec((_B // 128, 128), lambda i: (i, 0)),
        out_shape=jax.ShapeDtypeStruct((_P2 // 128, 128), jnp.float32),
    )(gu, gv, wp2, bp2)
    return out.reshape(_P2)


def kernel(x, edge_index, pos_edge_index, neg_edge_index, Wl1, bl1, Wr1, g1, b1,
           Wl2, bl2, Wr2, g2, b2, Wl3, bl3, Wr3, Wp1, bp1, Wp2, bp2, Wc1, bc1,
           Wc2, bc2):
    src = edge_index[0]
    dst = edge_index[1]
    srcp = jnp.concatenate([src, jnp.zeros((_EPAD - _E,), jnp.int32)])
    dstp = jnp.concatenate([dst, jnp.full((_EPAD - _E,), _NPAD - 1, jnp.int32)])
    src2 = jnp.stack([2 * srcp, 2 * srcp + 1])
    src2 = src2.reshape(_NC, _NS, _NCHUNKS, _CHUNK)
    dstp = dstp.reshape(_NS, _NCHUNKS, _CHUNK)

    zrows = jnp.zeros((_RPT, 64), jnp.float32)
    zcnt = jnp.zeros((_RPT, 16), jnp.float32)
    ones_r = jnp.ones((_CHUNK, 16), jnp.float32)

    agg1, cnt = _sc_seg_sum(_half(x), src2, dstp, zrows, zcnt, ones_r, True)
    c0 = cnt[:_N]

    hf, hb = _phase_a(
        x, agg1[0, :_N], agg1[1, :_N], c0, Wl1.T, bl1.reshape(1, -1),
        Wr1.T, g1.reshape(1, -1), b1.reshape(1, -1))

    agg2f, _ = _sc_seg_sum(_half(hf), src2, dstp, zrows, zcnt, ones_r, False)
    agg2b, _ = _sc_seg_sum(_half(hb), src2, dstp, zrows, zcnt, ones_r, False)

    yl, yr = _phase_b(
        hf, hb, agg2f[0, :_N], agg2f[1, :_N], agg2b[0, :_N], agg2b[1, :_N],
        c0, Wl2.T, bl2.reshape(1, -1), Wr2.T, g2.reshape(1, -1),
        b2.reshape(1, -1), Wl3.T, Wr3.T)

    agg3, _ = _sc_seg_sum(_half(yl), src2, dstp, zrows, zcnt, ones_r, False)

    z, node_logits, zu_proj, zv_proj = _phase_c(
        agg3[0, :_N], agg3[1, :_N], c0, yr, bl3.reshape(1, -1), Wc1.T,
        bc1.reshape(1, -1), Wc2.T, bc2.reshape(1, -1), Wp1[:, :128].T,
        Wp1[:, 128:].T, bp1.reshape(1, -1))

    u = jnp.concatenate([pos_edge_index[0], neg_edge_index[0]])
    v = jnp.concatenate([pos_edge_index[1], neg_edge_index[1]])
    u = u.reshape(_NC * _NS, _NCH_L, _CHUNK)
    v = v.reshape(_NC * _NS, _NCH_L, _CHUNK)
    gu, gv = _sc_link_gather(zu_proj, zv_proj, u, v)

    link_probs = _phase_d(gu, gv, Wp2, bp2.reshape(1, 1))
    return (z, node_logits, link_probs)


# speedup vs baseline: 1.2023x; 1.2023x over previous
import numpy as np
import jax
from jax import lax
import jax.numpy as jnp
from jax.experimental import pallas as pl
from jax.experimental.pallas import tpu as pltpu
from jax.experimental.pallas import tpu_sc as plsc

_N = 10000
_E = 320000
_P2 = 131072
_BN_S = np.float32(1.0 / np.sqrt(1.0 + 1e-5))

_NC, _NS = 2, 16
_NPAD = 10240
_EPAD = 327680
_CHUNK = 256
_RPT = _NPAD // _NS

_MESH = plsc.VectorSubcoreMesh(core_axis_name="c", subcore_axis_name="s",
                               num_cores=_NC, num_subcores=_NS)

_ECT = _EPAD // _NS
_NCHUNKS = _ECT // _CHUNK
_NPAIRS = _NCHUNKS // 2


def _sc_seg_sum(tab, srcp, dstp, zrows, zcnt, ones_r, with_cnt):
    def body(tar, srcr, dstr, zr, zc, onesr, out, outc,
             idxs, idxd, rows_a, rows_b, ones_v, acc, accc,
             semi, sem_a, sem_b):
        c = lax.axis_index("c")
        s = lax.axis_index("s")
        r0 = s * _RPT
        ci = pltpu.async_copy(srcr.at[c, s], idxs, semi)
        cd = pltpu.async_copy(dstr.at[s], idxd, semi)
        pltpu.sync_copy(zr, acc.at[pl.ds(r0, _RPT)])
        if with_cnt:
            @pl.when(c == 0)
            def _():
                pltpu.sync_copy(zc, accc.at[pl.ds(r0, _RPT)])
                pltpu.sync_copy(onesr, ones_v)
        ci.wait()
        cd.wait()

        def gather(k, rows, sem):
            pltpu.async_copy(tar.at[idxs.at[k]], rows, sem)

        def consume(k, rows, sem):
            pltpu.make_async_copy(tar.at[idxs.at[k]], rows, sem).wait()
            pltpu.sync_copy(rows, acc.at[idxd.at[k]], add=True)
            if with_cnt:
                @pl.when(c == 0)
                def _():
                    pltpu.sync_copy(ones_v, accc.at[idxd.at[k]], add=True)

        gather(0, rows_a, sem_a)
        plsc.subcore_barrier()

        @pl.loop(0, _NPAIRS)
        def _(p):
            ka = 2 * p
            gather(ka + 1, rows_b, sem_b)
            consume(ka, rows_a, sem_a)

            @pl.when(p + 1 < _NPAIRS)
            def _():
                gather(ka + 2, rows_a, sem_a)

            consume(ka + 1, rows_b, sem_b)

        plsc.subcore_barrier()

        @pl.loop(0, _RPT // 128)
        def _(i):
            rr = r0 + i * 128
            pltpu.sync_copy(acc.at[pl.ds(rr, 128)], rows_a.at[pl.ds(0, 128)])
            pltpu.sync_copy(rows_a.at[pl.ds(0, 128)], out.at[c, pl.ds(rr, 128)])

        if with_cnt:
            @pl.when(c == 0)
            def _():
                @pl.loop(0, _RPT // 128)
                def _(i):
                    rr = r0 + i * 128
                    pltpu.sync_copy(accc.at[pl.ds(rr, 128)],
                                    ones_v.at[pl.ds(0, 128)])
                    pltpu.sync_copy(ones_v.at[pl.ds(0, 128)],
                                    outc.at[pl.ds(rr, 128)])

    f = pl.kernel(
        body,
        out_type=(jax.ShapeDtypeStruct((_NC, _NPAD, 64), jnp.float32),
                  jax.ShapeDtypeStruct((_NPAD, 16), jnp.float32)),
        mesh=_MESH,
        compiler_params=pltpu.CompilerParams(use_tc_tiling_on_sc=False),
        scratch_types=[
            pltpu.VMEM((_NCHUNKS, _CHUNK), jnp.int32),
            pltpu.VMEM((_NCHUNKS, _CHUNK), jnp.int32),
            pltpu.VMEM((_CHUNK, 64), jnp.float32),
            pltpu.VMEM((_CHUNK, 64), jnp.float32),
            pltpu.VMEM((_CHUNK, 16), jnp.float32),
            pltpu.VMEM_SHARED((_NPAD, 64), jnp.float32),
            pltpu.VMEM_SHARED((_NPAD, 16), jnp.float32),
            pltpu.SemaphoreType.DMA,
            pltpu.SemaphoreType.DMA,
            pltpu.SemaphoreType.DMA,
        ],
    )
    return f(tab, srcp, dstp, zrows, zcnt, ones_r)


_EPT_L = _P2 // (_NC * _NS)
_NCH_L = _EPT_L // _CHUNK


def _sc_link_gather(zu, zv, uidx, vidx):
    def body(tu, tv, ur, vr, gu, gv, idx_u, idx_v, rows_u, rows_v,
             semi, sem_u, sem_v):
        c = lax.axis_index("c")
        s = lax.axis_index("s")
        t = c * _NS + s
        base = t * _EPT_L
        cu = pltpu.async_copy(ur.at[t], idx_u, semi)
        cv = pltpu.async_copy(vr.at[t], idx_v, semi)
        cu.wait()
        cv.wait()
        pltpu.async_copy(tu.at[idx_u.at[0]], rows_u, sem_u)

        @pl.loop(0, _NCH_L)
        def _(k):
            off = base + k * _CHUNK
            pltpu.async_copy(tv.at[idx_v.at[k]], rows_v, sem_v)
            pltpu.make_async_copy(tu.at[idx_u.at[k]], rows_u, sem_u).wait()
            pltpu.sync_copy(rows_u, gu.at[pl.ds(off, _CHUNK)])

            @pl.when(k + 1 < _NCH_L)
            def _():
                pltpu.async_copy(tu.at[idx_u.at[k + 1]], rows_u, sem_u)

            pltpu.make_async_copy(tv.at[idx_v.at[k]], rows_v, sem_v).wait()
            pltpu.sync_copy(rows_v, gv.at[pl.ds(off, _CHUNK)])

    f = pl.kernel(
        body,
        out_type=(jax.ShapeDtypeStruct((_P2, 128), jnp.float32),
                  jax.ShapeDtypeStruct((_P2, 128), jnp.float32)),
        mesh=_MESH,
        compiler_params=pltpu.CompilerParams(use_tc_tiling_on_sc=False),
        scratch_types=[
            pltpu.VMEM((_NCH_L, _CHUNK), jnp.int32),
            pltpu.VMEM((_NCH_L, _CHUNK), jnp.int32),
            pltpu.VMEM((_CHUNK, 128), jnp.float32),
            pltpu.VMEM((_CHUNK, 128), jnp.float32),
            pltpu.SemaphoreType.DMA,
            pltpu.SemaphoreType.DMA,
            pltpu.SemaphoreType.DMA,
        ],
    )
    return f(zu, zv, uidx, vidx)


def _cnt_col(cr):
    return jnp.maximum(cr[:, :1], 1.0)


_RB = 2000
_NRB = _N // _RB


def _row_call(body, n_rowed, weight_shapes, out_cols):
    def block(c):
        return pl.BlockSpec((_RB, c), lambda i: (i, 0))

    def full(shape):
        return pl.BlockSpec(shape, lambda i: (0, 0))

    def outspec(c):
        if c == -1:
            return pl.BlockSpec((2, _RB, 64), lambda i: (0, i, 0))
        return block(c)

    def outshape(c):
        if c == -1:
            return jax.ShapeDtypeStruct((2, _N, 64), jnp.float32)
        return jax.ShapeDtypeStruct((_N, c), jnp.float32)

    def make(*arrays):
        in_specs = [block(a.shape[1]) for a in arrays[:n_rowed]]
        in_specs += [full(s) for s in weight_shapes]
        return pl.pallas_call(
            body,
            grid=(_NRB,),
            in_specs=in_specs,
            out_specs=tuple(outspec(c) for c in out_cols),
            out_shape=tuple(outshape(c) for c in out_cols),
        )(*arrays)
    return make


def _phase_a(x, a0, a1, c0, Wl1T, bl1, Wr1T, g1, b1):
    def body(x_ref, a0r, a1r, c0r, wl_ref, blr, wr_ref, gr, br, hf, hb):
        mean = jnp.concatenate([a0r[...], a1r[...]], axis=-1) / _cnt_col(c0r[...])
        h = jnp.dot(mean, wl_ref[...], preferred_element_type=jnp.float32)
        h = h + jnp.dot(x_ref[...], wr_ref[...], preferred_element_type=jnp.float32)
        h = (h + blr[...]) * (gr[...] * _BN_S) + br[...]
        h = jnp.maximum(h, 0.0)
        hf[0] = h[:, :64]
        hf[1] = h[:, 64:128]
        hb[0] = h[:, 128:192]
        hb[1] = h[:, 192:]

    wts = [Wl1T, bl1, Wr1T, g1, b1]
    return _row_call(body, 4, [w.shape for w in wts], (-1, -1))(
        x, a0, a1, c0, *wts)


def _phase_b(hf0, hf1, hb0, hb1, f0, f1, b0, b1, c0, Wl2T, bl2, Wr2T, g2, b2,
             Wl3T, Wr3T):
    def body(hf0r, hf1r, hb0r, hb1r, f0r, f1r, b0r, b1r, c0r, wl, blr, wr,
             gr, br, wl3, wr3, yl_ref, yr_ref):
        cnt = _cnt_col(c0r[...])
        mean = jnp.concatenate(
            [f0r[...], f1r[...], b0r[...], b1r[...]], axis=-1) / cnt
        h1 = jnp.concatenate(
            [hf0r[...], hf1r[...], hb0r[...], hb1r[...]], axis=-1)
        h = jnp.dot(mean, wl[...], preferred_element_type=jnp.float32)
        h = h + jnp.dot(h1, wr[...], preferred_element_type=jnp.float32)
        h = (h + blr[...]) * (gr[...] * _BN_S) + br[...]
        h = jnp.maximum(h, 0.0)
        yl = jnp.dot(h, wl3[...], preferred_element_type=jnp.float32)
        yl_ref[0] = yl[:, :64]
        yl_ref[1] = yl[:, 64:]
        yr_ref[...] = jnp.dot(h, wr3[...], preferred_element_type=jnp.float32)

    wts = [Wl2T, bl2, Wr2T, g2, b2, Wl3T, Wr3T]
    return _row_call(body, 9, [w.shape for w in wts], (-1, 128))(
        hf0, hf1, hb0, hb1, f0, f1, b0, b1, c0, *wts)


def _phase_c(a0, a1, c0, yr, bl3, Wc1T, bc1, Wc2T, bc2, Wp1aT, Wp1bT, bp1):
    def body(a0r, a1r, c0r, yr_ref, bl3r, wc1, bc1r, wc2, bc2r, wpa, wpb,
             bp1r, z_ref, nl_ref, zu_ref, zv_ref):
        mean = jnp.concatenate([a0r[...], a1r[...]], axis=-1) / _cnt_col(c0r[...])
        z = mean + bl3r[...] + yr_ref[...]
        z_ref[...] = z
        t = jnp.maximum(jnp.dot(z, wc1[...], preferred_element_type=jnp.float32)
                        + bc1r[...], 0.0)
        nl_ref[...] = jnp.dot(t, wc2[...], preferred_element_type=jnp.float32) + bc2r[...]
        zu_ref[...] = jnp.dot(z, wpa[...], preferred_element_type=jnp.float32) + bp1r[...]
        zv_ref[...] = jnp.dot(z, wpb[...], preferred_element_type=jnp.float32)

    wts = [bl3, Wc1T, bc1, Wc2T, bc2, Wp1aT, Wp1bT, bp1]
    return _row_call(body, 4, [w.shape for w in wts], (128, 4, 128, 128))(
        a0, a1, c0, yr, *wts)


def _phase_d(gu, gv, wp2, bp2):
    _NB = 16
    _B = _P2 // _NB

    def body(gu_ref, gv_ref, w_ref, b_ref, out_ref):
        e = jnp.maximum(gu_ref[...] + gv_ref[...], 0.0)
        logit = jnp.sum(e * w_ref[...], axis=-1) + b_ref[0, 0]
        out_ref[...] = jax.nn.sigmoid(logit).reshape(_B // 128, 128)

    out = pl.pallas_call(
        body,
        grid=(_NB,),
        in_specs=[pl.BlockSpec((_B, 128), lambda i: (i, 0)),
                  pl.BlockSpec((_B, 128), lambda i: (i, 0)),
                  pl.BlockSpec((1, 128), lambda i: (0, 0)),
                  pl.BlockSpec((1, 1), lambda i: (0, 0))],
        out_specs=pl.BlockSpec((_B // 128, 128), lambda i: (i, 0)),
        out_shape=jax.ShapeDtypeStruct((_P2 // 128, 128), jnp.float32),
    )(gu, gv, wp2, bp2)
    return out.reshape(_P2)


def kernel(x, edge_index, pos_edge_index, neg_edge_index, Wl1, bl1, Wr1, g1, b1,
           Wl2, bl2, Wr2, g2, b2, Wl3, bl3, Wr3, Wp1, bp1, Wp2, bp2, Wc1, bc1,
           Wc2, bc2):
    src = edge_index[0]
    dst = edge_index[1]
    srcp = jnp.concatenate([src, jnp.zeros((_EPAD - _E,), jnp.int32)])
    dstp = jnp.concatenate([dst, jnp.full((_EPAD - _E,), _NPAD - 1, jnp.int32)])
    src2 = jnp.stack([srcp, srcp + _N])
    src2 = src2.reshape(_NC, _NS, _NCHUNKS, _CHUNK)
    dstp = dstp.reshape(_NS, _NCHUNKS, _CHUNK)

    zrows = jnp.zeros((_RPT, 64), jnp.float32)
    zcnt = jnp.zeros((_RPT, 16), jnp.float32)
    ones_r = jnp.ones((_CHUNK, 16), jnp.float32)

    x2 = jnp.concatenate([x[:, :64], x[:, 64:]], axis=0)
    agg1, cnt = _sc_seg_sum(x2, src2, dstp, zrows, zcnt, ones_r, True)
    c0 = cnt[:_N]

    hfs, hbs = _phase_a(
        x, agg1[0, :_N], agg1[1, :_N], c0, Wl1.T, bl1.reshape(1, -1),
        Wr1.T, g1.reshape(1, -1), b1.reshape(1, -1))

    agg2f, _ = _sc_seg_sum(hfs.reshape(2 * _N, 64), src2, dstp, zrows, zcnt,
                           ones_r, False)
    agg2b, _ = _sc_seg_sum(hbs.reshape(2 * _N, 64), src2, dstp, zrows, zcnt,
                           ones_r, False)

    yls, yr = _phase_b(
        hfs[0], hfs[1], hbs[0], hbs[1],
        agg2f[0, :_N], agg2f[1, :_N], agg2b[0, :_N], agg2b[1, :_N],
        c0, Wl2.T, bl2.reshape(1, -1), Wr2.T, g2.reshape(1, -1),
        b2.reshape(1, -1), Wl3.T, Wr3.T)

    agg3, _ = _sc_seg_sum(yls.reshape(2 * _N, 64), src2, dstp, zrows, zcnt,
                          ones_r, False)

    z, node_logits, zu_proj, zv_proj = _phase_c(
        agg3[0, :_N], agg3[1, :_N], c0, yr, bl3.reshape(1, -1), Wc1.T,
        bc1.reshape(1, -1), Wc2.T, bc2.reshape(1, -1), Wp1[:, :128].T,
        Wp1[:, 128:].T, bp1.reshape(1, -1))

    u = jnp.concatenate([pos_edge_index[0], neg_edge_index[0]])
    v = jnp.concatenate([pos_edge_index[1], neg_edge_index[1]])
    u = u.reshape(_NC * _NS, _NCH_L, _CHUNK)
    v = v.reshape(_NC * _NS, _NCH_L, _CHUNK)
    gu, gv = _sc_link_gather(zu_proj, zv_proj, u, v)

    link_probs = _phase_d(gu, gv, Wp2, bp2.reshape(1, 1))
    return (z, node_logits, link_probs)
